# Initial kernel scaffold; baseline (speedup 1.0000x reference)
import functools

import jax
import jax.numpy as jnp
from jax import lax
from jax.experimental import pallas as pl
from jax.experimental.pallas import tpu as pltpu
from jax.experimental.pallas import tpu_sc as plsc

_NC, _NS = 2, 16
_NW = _NC * _NS
_K = 128
_B = 512


def _sc_gather(idx_a, idx_b, table, charge2d):
    nwng, k = idx_a.shape
    ng = nwng // _NW
    _, e1 = table.shape
    npad = nwng * k

    @functools.partial(
        pl.kernel,
        mesh=plsc.VectorSubcoreMesh(core_axis_name="c", subcore_axis_name="s"),
        out_type=[
            jax.ShapeDtypeStruct((npad, e1), jnp.float32),
            jax.ShapeDtypeStruct((npad, 1), jnp.int32),
        ],
        scratch_types=[
            pltpu.VMEM((ng, k), jnp.int32),
            pltpu.VMEM((ng, k), jnp.int32),
            pltpu.VMEM((k, e1), jnp.float32),
            pltpu.VMEM((k, 1), jnp.int32),
            pltpu.SemaphoreType.DMA,
            pltpu.SemaphoreType.DMA,
        ],
    )
    def sc_k(idx_a_hbm, idx_b_hbm, table_hbm, charge_hbm, ea_hbm, cn_hbm,
             ia_v, ib_v, rows_v, cv_v, gsem, csem):
        wid = lax.axis_index("s") * _NC + lax.axis_index("c")
        pltpu.sync_copy(idx_a_hbm.at[pl.ds(wid * ng, ng)], ia_v)
        pltpu.sync_copy(idx_b_hbm.at[pl.ds(wid * ng, ng)], ib_v)
        base = wid * ng * k
        for g in range(ng):
            ga = pltpu.async_copy(table_hbm.at[ia_v.at[g]], rows_v, gsem)
            gc = pltpu.async_copy(charge_hbm.at[ib_v.at[g]], cv_v, csem)
            ga.wait()
            gc.wait()
            pltpu.sync_copy(rows_v, ea_hbm.at[pl.ds(base + g * k, k)])
            pltpu.sync_copy(cv_v, cn_hbm.at[pl.ds(base + g * k, k)])

    return sc_k(idx_a, idx_b, table, charge2d)


def _tc_body(ea_ref, pos_ref, cn_ref, w1_ref, b1_ref, w2_ref, b2_ref,
             ec_ref, wa_ref, wh_ref, wc_ref, o_ref):
    pos = pos_ref[...]
    t = jnp.dot(pos, w1_ref[...], preferred_element_type=jnp.float32) + b1_ref[...]
    t = t * jax.nn.sigmoid(t)
    h = jnp.dot(t, w2_ref[...], preferred_element_type=jnp.float32) + b2_ref[...]
    cn = cn_ref[...]
    ncls = ec_ref.shape[0]
    oh = (cn == lax.broadcasted_iota(jnp.int32, (cn.shape[0], ncls), 1))
    ech = jnp.dot(oh.astype(jnp.float32), ec_ref[...],
                  preferred_element_type=jnp.float32)
    acc = (jnp.dot(ea_ref[...], wa_ref[...], preferred_element_type=jnp.float32)
           + jnp.dot(h, wh_ref[...], preferred_element_type=jnp.float32)
           + jnp.dot(ech, wc_ref[...], preferred_element_type=jnp.float32))
    o_ref[...] = acc * jax.nn.sigmoid(acc)


def kernel(batch, atom_type, pos_feat, charge, emb_atom, W1, b1, W2, b2,
           emb_charge, W_proj):
    n = batch.shape[0]
    v, e1 = emb_atom.shape
    in_dim, e2 = W1.shape
    vc, e3 = emb_charge.shape
    out_dim = W_proj.shape[1]

    batch = batch.astype(jnp.int32)
    atom_type = atom_type.astype(jnp.int32)
    charge = charge.astype(jnp.int32)

    ng = -(-n // (_NW * _K))
    npad = _NW * ng * _K
    pad = npad - n
    idx_a = jnp.pad(atom_type, (0, pad)).reshape(_NW * ng, _K)
    idx_b = jnp.pad(batch, (0, pad)).reshape(_NW * ng, _K)

    ea, cn = _sc_gather(idx_a, idx_b, emb_atom, charge[:, None])

    ncls = 32
    ec_pad = jnp.pad(emb_charge, ((0, ncls - vc), (0, 0)))
    wa = W_proj[:e1]
    wh = W_proj[e1:e1 + e2]
    wc = W_proj[e1 + e2:]
    n_blocks = -(-n // _B)

    out = pl.pallas_call(
        _tc_body,
        grid=(n_blocks,),
        in_specs=[
            pl.BlockSpec((_B, e1), lambda i: (i, 0)),
            pl.BlockSpec((_B, in_dim), lambda i: (i, 0)),
            pl.BlockSpec((_B, 1), lambda i: (i, 0)),
            pl.BlockSpec((in_dim, e2), lambda i: (0, 0)),
            pl.BlockSpec((1, e2), lambda i: (0, 0)),
            pl.BlockSpec((e2, e2), lambda i: (0, 0)),
            pl.BlockSpec((1, e2), lambda i: (0, 0)),
            pl.BlockSpec((ncls, e3), lambda i: (0, 0)),
            pl.BlockSpec((e1, out_dim), lambda i: (0, 0)),
            pl.BlockSpec((e2, out_dim), lambda i: (0, 0)),
            pl.BlockSpec((e3, out_dim), lambda i: (0, 0)),
        ],
        out_specs=pl.BlockSpec((_B, out_dim), lambda i: (i, 0)),
        out_shape=jax.ShapeDtypeStruct((n, out_dim), jnp.float32),
    )(ea, pos_feat, cn, W1, b1[None, :], W2, b2[None, :], ec_pad, wa, wh, wc)
    return out


# speedup vs baseline: 1.9351x; 1.9351x over previous
import functools

import jax
import jax.numpy as jnp
from jax import lax
from jax.experimental import pallas as pl
from jax.experimental.pallas import tpu as pltpu
from jax.experimental.pallas import tpu_sc as plsc

_NC, _NS = 2, 16
_NW = _NC * _NS
_K = 128
_B = 512


_CW = 16


def _sc_gather(idx_a, idx_b, table, charge2d):
    nw, ng, k = idx_a.shape
    _, e1 = table.shape
    npad = nw * ng * k

    @functools.partial(
        pl.kernel,
        mesh=plsc.VectorSubcoreMesh(core_axis_name="c", subcore_axis_name="s"),
        compiler_params=pltpu.CompilerParams(use_tc_tiling_on_sc=False),
        out_type=[
            jax.ShapeDtypeStruct((npad, e1), jnp.float32),
            jax.ShapeDtypeStruct((npad, _CW), jnp.int32),
        ],
        scratch_types=[
            pltpu.VMEM((ng, k), jnp.int32),
            pltpu.VMEM((ng, k), jnp.int32),
            pltpu.VMEM((k, e1), jnp.float32),
            pltpu.VMEM((k, _CW), jnp.int32),
            pltpu.SemaphoreType.DMA,
            pltpu.SemaphoreType.DMA,
        ],
    )
    def sc_k(idx_a_hbm, idx_b_hbm, table_hbm, charge_hbm, ea_hbm, cn_hbm,
             ia_v, ib_v, rows_v, cv_v, gsem, csem):
        wid = lax.axis_index("s") * _NC + lax.axis_index("c")
        pltpu.sync_copy(idx_a_hbm.at[wid], ia_v)
        pltpu.sync_copy(idx_b_hbm.at[wid], ib_v)
        base = wid * ng * k
        for g in range(ng):
            ga = pltpu.async_copy(table_hbm.at[ia_v.at[g]], rows_v, gsem)
            gc = pltpu.async_copy(charge_hbm.at[ib_v.at[g]], cv_v, csem)
            ga.wait()
            gc.wait()
            pltpu.sync_copy(rows_v, ea_hbm.at[pl.ds(base + g * k, k)])
            pltpu.sync_copy(cv_v, cn_hbm.at[pl.ds(base + g * k, k)])

    return sc_k(idx_a, idx_b, table, charge2d)


def _tc_body(ea_ref, pos_ref, cn_ref, w1_ref, b1_ref, w2_ref, b2_ref,
             ec_ref, wa_ref, wh_ref, wc_ref, o_ref):
    pos = pos_ref[...]
    t = jnp.dot(pos, w1_ref[...], preferred_element_type=jnp.float32) + b1_ref[...]
    t = t * jax.nn.sigmoid(t)
    h = jnp.dot(t, w2_ref[...], preferred_element_type=jnp.float32) + b2_ref[...]
    cn = cn_ref[:, :1]
    ncls = ec_ref.shape[0]
    oh = (cn == lax.broadcasted_iota(jnp.int32, (cn.shape[0], ncls), 1))
    ech = jnp.dot(oh.astype(jnp.float32), ec_ref[...],
                  preferred_element_type=jnp.float32)
    acc = (jnp.dot(ea_ref[...], wa_ref[...], preferred_element_type=jnp.float32)
           + jnp.dot(h, wh_ref[...], preferred_element_type=jnp.float32)
           + jnp.dot(ech, wc_ref[...], preferred_element_type=jnp.float32))
    o_ref[...] = acc * jax.nn.sigmoid(acc)


def kernel(batch, atom_type, pos_feat, charge, emb_atom, W1, b1, W2, b2,
           emb_charge, W_proj):
    n = batch.shape[0]
    v, e1 = emb_atom.shape
    in_dim, e2 = W1.shape
    vc, e3 = emb_charge.shape
    out_dim = W_proj.shape[1]

    batch = batch.astype(jnp.int32)
    atom_type = atom_type.astype(jnp.int32)
    charge = charge.astype(jnp.int32)

    ng = -(-n // (_NW * _K))
    npad = _NW * ng * _K
    pad = npad - n
    idx_a = jnp.pad(atom_type, (0, pad)).reshape(_NW, ng, _K)
    idx_b = jnp.pad(batch, (0, pad)).reshape(_NW, ng, _K)

    g = charge.shape[0]
    charge_rows = jnp.broadcast_to(charge[:, None], (g, _CW))
    ea, cn = _sc_gather(idx_a, idx_b, emb_atom, charge_rows)

    ncls = 32
    ec_pad = jnp.pad(emb_charge, ((0, ncls - vc), (0, 0)))
    wa = W_proj[:e1]
    wh = W_proj[e1:e1 + e2]
    wc = W_proj[e1 + e2:]
    n_blocks = -(-n // _B)

    out = pl.pallas_call(
        _tc_body,
        grid=(n_blocks,),
        in_specs=[
            pl.BlockSpec((_B, e1), lambda i: (i, 0)),
            pl.BlockSpec((_B, in_dim), lambda i: (i, 0)),
            pl.BlockSpec((_B, _CW), lambda i: (i, 0)),
            pl.BlockSpec((in_dim, e2), lambda i: (0, 0)),
            pl.BlockSpec((1, e2), lambda i: (0, 0)),
            pl.BlockSpec((e2, e2), lambda i: (0, 0)),
            pl.BlockSpec((1, e2), lambda i: (0, 0)),
            pl.BlockSpec((ncls, e3), lambda i: (0, 0)),
            pl.BlockSpec((e1, out_dim), lambda i: (0, 0)),
            pl.BlockSpec((e2, out_dim), lambda i: (0, 0)),
            pl.BlockSpec((e3, out_dim), lambda i: (0, 0)),
        ],
        out_specs=pl.BlockSpec((_B, out_dim), lambda i: (i, 0)),
        out_shape=jax.ShapeDtypeStruct((n, out_dim), jnp.float32),
    )(ea, pos_feat, cn, W1, b1[None, :], W2, b2[None, :], ec_pad, wa, wh, wc)
    return out


# speedup vs baseline: 1.9895x; 1.0281x over previous
import functools

import jax
import jax.numpy as jnp
from jax import lax
from jax.experimental import pallas as pl
from jax.experimental.pallas import tpu as pltpu
from jax.experimental.pallas import tpu_sc as plsc

_NC, _NS = 2, 16
_NW = _NC * _NS
_K = 128
_B = 512


_CW = 16


def _sc_gather(idx_a, idx_b, table, charge2d):
    nw, ng, k = idx_a.shape
    _, e1 = table.shape
    npad = nw * ng * k

    nbuf = 4

    @functools.partial(
        pl.kernel,
        mesh=plsc.VectorSubcoreMesh(core_axis_name="c", subcore_axis_name="s"),
        compiler_params=pltpu.CompilerParams(use_tc_tiling_on_sc=False),
        out_type=[
            jax.ShapeDtypeStruct((npad, e1), jnp.float32),
            jax.ShapeDtypeStruct((npad, _CW), jnp.int32),
        ],
        scratch_types=(
            [pltpu.VMEM((ng, k), jnp.int32),
             pltpu.VMEM((ng, k), jnp.int32),
             pltpu.VMEM((nbuf, k, e1), jnp.float32),
             pltpu.VMEM((nbuf, k, _CW), jnp.int32)]
            + [pltpu.SemaphoreType.DMA] * (2 * nbuf)
        ),
    )
    def sc_k(idx_a_hbm, idx_b_hbm, table_hbm, charge_hbm, ea_hbm, cn_hbm,
             ia_v, ib_v, rows_v, cv_v, *sems):
        gsems, ssems = sems[:nbuf], sems[nbuf:]
        wid = lax.axis_index("s") * _NC + lax.axis_index("c")
        pltpu.sync_copy(idx_a_hbm.at[wid], ia_v)
        pltpu.sync_copy(idx_b_hbm.at[wid], ib_v)
        base = wid * ng * k

        def start_gather(g):
            b = g % nbuf
            return (
                pltpu.async_copy(table_hbm.at[ia_v.at[g]], rows_v.at[b], gsems[b]),
                pltpu.async_copy(charge_hbm.at[ib_v.at[g]], cv_v.at[b], gsems[b]),
            )

        def start_store(g):
            b = g % nbuf
            return (
                pltpu.async_copy(rows_v.at[b], ea_hbm.at[pl.ds(base + g * k, k)], ssems[b]),
                pltpu.async_copy(cv_v.at[b], cn_hbm.at[pl.ds(base + g * k, k)], ssems[b]),
            )

        gd = [None] * nbuf
        sd = [None] * nbuf
        issued = 0
        for g in range(ng):
            while issued < min(g + nbuf, ng):
                b = issued % nbuf
                if sd[b] is not None:
                    for d in sd[b]:
                        d.wait()
                    sd[b] = None
                gd[b] = start_gather(issued)
                issued += 1
            b = g % nbuf
            for d in gd[b]:
                d.wait()
            sd[b] = start_store(g)
        for b in range(nbuf):
            if sd[b] is not None:
                for d in sd[b]:
                    d.wait()

    return sc_k(idx_a, idx_b, table, charge2d)


def _tc_body(ea_ref, pos_ref, cn_ref, w1_ref, b1_ref, w2_ref, b2_ref,
             ec_ref, wa_ref, wh_ref, wc_ref, o_ref):
    pos = pos_ref[...]
    t = jnp.dot(pos, w1_ref[...], preferred_element_type=jnp.float32) + b1_ref[...]
    t = t * jax.nn.sigmoid(t)
    h = jnp.dot(t, w2_ref[...], preferred_element_type=jnp.float32) + b2_ref[...]
    cn = cn_ref[:, :1]
    ncls = ec_ref.shape[0]
    oh = (cn == lax.broadcasted_iota(jnp.int32, (cn.shape[0], ncls), 1))
    ech = jnp.dot(oh.astype(jnp.float32), ec_ref[...],
                  preferred_element_type=jnp.float32)
    acc = (jnp.dot(ea_ref[...], wa_ref[...], preferred_element_type=jnp.float32)
           + jnp.dot(h, wh_ref[...], preferred_element_type=jnp.float32)
           + jnp.dot(ech, wc_ref[...], preferred_element_type=jnp.float32))
    o_ref[...] = acc * jax.nn.sigmoid(acc)


def kernel(batch, atom_type, pos_feat, charge, emb_atom, W1, b1, W2, b2,
           emb_charge, W_proj):
    n = batch.shape[0]
    v, e1 = emb_atom.shape
    in_dim, e2 = W1.shape
    vc, e3 = emb_charge.shape
    out_dim = W_proj.shape[1]

    batch = batch.astype(jnp.int32)
    atom_type = atom_type.astype(jnp.int32)
    charge = charge.astype(jnp.int32)

    ng = -(-n // (_NW * _K))
    npad = _NW * ng * _K
    pad = npad - n
    idx_a = jnp.pad(atom_type, (0, pad)).reshape(_NW, ng, _K)
    idx_b = jnp.pad(batch, (0, pad)).reshape(_NW, ng, _K)

    g = charge.shape[0]
    charge_rows = jnp.broadcast_to(charge[:, None], (g, _CW))
    ea, cn = _sc_gather(idx_a, idx_b, emb_atom, charge_rows)

    ncls = 32
    ec_pad = jnp.pad(emb_charge, ((0, ncls - vc), (0, 0)))
    wa = W_proj[:e1]
    wh = W_proj[e1:e1 + e2]
    wc = W_proj[e1 + e2:]
    n_blocks = -(-n // _B)

    out = pl.pallas_call(
        _tc_body,
        grid=(n_blocks,),
        in_specs=[
            pl.BlockSpec((_B, e1), lambda i: (i, 0)),
            pl.BlockSpec((_B, in_dim), lambda i: (i, 0)),
            pl.BlockSpec((_B, _CW), lambda i: (i, 0)),
            pl.BlockSpec((in_dim, e2), lambda i: (0, 0)),
            pl.BlockSpec((1, e2), lambda i: (0, 0)),
            pl.BlockSpec((e2, e2), lambda i: (0, 0)),
            pl.BlockSpec((1, e2), lambda i: (0, 0)),
            pl.BlockSpec((ncls, e3), lambda i: (0, 0)),
            pl.BlockSpec((e1, out_dim), lambda i: (0, 0)),
            pl.BlockSpec((e2, out_dim), lambda i: (0, 0)),
            pl.BlockSpec((e3, out_dim), lambda i: (0, 0)),
        ],
        out_specs=pl.BlockSpec((_B, out_dim), lambda i: (i, 0)),
        out_shape=jax.ShapeDtypeStruct((n, out_dim), jnp.float32),
    )(ea, pos_feat, cn, W1, b1[None, :], W2, b2[None, :], ec_pad, wa, wh, wc)
    return out
